# Initial kernel scaffold; baseline (speedup 1.0000x reference)
import jax
import jax.numpy as jnp
from jax.experimental import pallas as pl


def kernel(feature, segment_ids):
    raise NotImplementedError("write your pallas kernel here")


# speedup vs baseline: 3.9088x; 3.9088x over previous
import functools

import jax
import jax.numpy as jnp
from jax import lax
from jax.experimental import pallas as pl
from jax.experimental.pallas import tpu as pltpu, tpu_sc as plsc

N = 50000
D = 256
NUM_GRAPHS = 512

NC = 2
NS = 16
NW = NC * NS
NPAD = 50176
CHUNK = NPAD // NW
CVECS = CHUNK // 16
HCHUNK = NPAD // NS
HVECS = HCHUNK // 16
BINS = 640
PAD_ID = NUM_GRAPHS

_sc_mesh = plsc.VectorSubcoreMesh(core_axis_name="c", subcore_axis_name="s")


@functools.partial(
    pl.kernel,
    out_type=jax.ShapeDtypeStruct((NPAD,), jnp.float32),
    mesh=_sc_mesh,
    scratch_types=[
        pltpu.VMEM((HCHUNK,), jnp.int32),
        pltpu.VMEM((CHUNK,), jnp.int32),
        pltpu.VMEM((BINS,), jnp.float32),
        pltpu.VMEM((BINS // 128, 128), jnp.int32),
        pltpu.VMEM((CHUNK,), jnp.float32),
        pltpu.VMEM_SHARED((BINS,), jnp.float32),
        pltpu.SemaphoreType.DMA,
    ],
    compiler_params=pltpu.CompilerParams(needs_layout_passes=False),
)
def _sc_counts(ids_hbm, out_hbm, idxh_v, idxo_v, hist_v, binidx_v, cnt_v,
               shared, sem):
    cid = lax.axis_index("c")
    sid = lax.axis_index("s")
    wid = sid * NC + cid
    pltpu.sync_copy(ids_hbm.at[pl.ds(sid * HCHUNK, HCHUNK)], idxh_v)
    pltpu.sync_copy(ids_hbm.at[pl.ds(wid * CHUNK, CHUNK)], idxo_v)

    zeros16 = jnp.zeros((16,), jnp.float32)
    ones16 = jnp.ones((16,), jnp.float32)
    iota16 = lax.iota(jnp.int32, 16)
    for j in range(BINS // 16):
        hist_v[pl.ds(j * 16, 16)] = zeros16
    for c in range(BINS // 128):
        for k in range(8):
            binidx_v[c, pl.ds(k * 16, 16)] = iota16 + (c * 128 + k * 16)

    @pl.when(sid == 0)
    def _():
        pltpu.sync_copy(hist_v, shared)

    for i in range(HVECS):
        v = idxh_v[pl.ds(i * 16, 16)]
        plsc.addupdate_scatter(hist_v, [v], ones16)

    plsc.subcore_barrier()
    for c in range(BINS // 128):
        pltpu.sync_copy(
            hist_v.at[pl.ds(c * 128, 128)],
            shared.at[binidx_v.at[c]],
            add=True,
        )
    plsc.subcore_barrier()
    pltpu.sync_copy(shared, hist_v)

    for i in range(CVECS):
        sl = pl.ds(i * 16, 16)
        cnt_v[sl] = plsc.load_gather(hist_v, [idxo_v[sl]])
    pltpu.sync_copy(cnt_v, out_hbm.at[pl.ds(wid * CHUNK, CHUNK)])


_BR = 1000


def _tc_scale_body(f_ref, c_ref, o_ref):
    o_ref[...] = f_ref[...] * lax.rsqrt(c_ref[...])


_tc_scale = pl.pallas_call(
    _tc_scale_body,
    grid=(N // _BR,),
    in_specs=[
        pl.BlockSpec((_BR, D), lambda k: (k, 0)),
        pl.BlockSpec((_BR, 1), lambda k: (k, 0)),
    ],
    out_specs=pl.BlockSpec((_BR, D), lambda k: (k, 0)),
    out_shape=jax.ShapeDtypeStruct((N, D), jnp.float32),
)


def kernel(feature, segment_ids):
    ids = segment_ids.astype(jnp.int32)
    ids_pad = jnp.concatenate(
        [ids, jnp.full((NPAD - N,), PAD_ID, jnp.int32)]
    )
    counts = _sc_counts(ids_pad)[:N]
    return _tc_scale(feature, counts[:, None])


# speedup vs baseline: 4.4762x; 1.1452x over previous
import functools

import jax
import jax.numpy as jnp
from jax import lax
from jax.experimental import pallas as pl
from jax.experimental.pallas import tpu as pltpu, tpu_sc as plsc

N = 50000
D = 256
NUM_GRAPHS = 512

NC = 2
NS = 16
NW = NC * NS
NPAD = 50176
CHUNK = NPAD // NW
CVECS = CHUNK // 16
HCHUNK = NPAD // NS
HVECS = HCHUNK // 16
BINS = 640
PAD_ID = NUM_GRAPHS

_sc_mesh = plsc.VectorSubcoreMesh(core_axis_name="c", subcore_axis_name="s")


@functools.partial(
    pl.kernel,
    out_type=jax.ShapeDtypeStruct((NPAD,), jnp.float32),
    mesh=_sc_mesh,
    scratch_types=[
        pltpu.VMEM((HCHUNK,), jnp.int32),
        pltpu.VMEM((CHUNK,), jnp.int32),
        pltpu.VMEM((BINS,), jnp.float32),
        pltpu.VMEM((BINS // 128, 128), jnp.int32),
        pltpu.VMEM((CHUNK,), jnp.float32),
        pltpu.VMEM_SHARED((BINS,), jnp.float32),
        pltpu.SemaphoreType.DMA,
    ],
    compiler_params=pltpu.CompilerParams(needs_layout_passes=False),
)
def _sc_counts(ids_hbm, out_hbm, idxh_v, idxo_v, hist_v, binidx_v, cnt_v,
               shared, sem):
    cid = lax.axis_index("c")
    sid = lax.axis_index("s")
    wid = sid * NC + cid
    pltpu.sync_copy(ids_hbm.at[pl.ds(sid * HCHUNK, HCHUNK)], idxh_v)
    pltpu.sync_copy(ids_hbm.at[pl.ds(wid * CHUNK, CHUNK)], idxo_v)

    zeros16 = jnp.zeros((16,), jnp.float32)
    ones16 = jnp.ones((16,), jnp.float32)
    iota16 = lax.iota(jnp.int32, 16)
    for j in range(BINS // 16):
        hist_v[pl.ds(j * 16, 16)] = zeros16
    for c in range(BINS // 128):
        for k in range(8):
            binidx_v[c, pl.ds(k * 16, 16)] = iota16 + (c * 128 + k * 16)

    @pl.when(sid == 0)
    def _():
        pltpu.sync_copy(hist_v, shared)

    for i in range(HVECS):
        v = idxh_v[pl.ds(i * 16, 16)]
        plsc.addupdate_scatter(hist_v, [v], ones16)

    plsc.subcore_barrier()
    for c in range(BINS // 128):
        pltpu.sync_copy(
            hist_v.at[pl.ds(c * 128, 128)],
            shared.at[binidx_v.at[c]],
            add=True,
        )
    plsc.subcore_barrier()
    pltpu.sync_copy(shared, hist_v)

    for i in range(CVECS):
        sl = pl.ds(i * 16, 16)
        cnt_v[sl] = plsc.load_gather(hist_v, [idxo_v[sl]])
    pltpu.sync_copy(cnt_v, out_hbm.at[pl.ds(wid * CHUNK, CHUNK)])


_BR = 2000


def _tc_scale_body(f_ref, c_ref, o_ref):
    o_ref[...] = f_ref[...] * lax.rsqrt(c_ref[...])


_tc_scale = pl.pallas_call(
    _tc_scale_body,
    grid=(N // _BR,),
    in_specs=[
        pl.BlockSpec((_BR, D), lambda k: (k, 0)),
        pl.BlockSpec((_BR, 1), lambda k: (k, 0)),
    ],
    out_specs=pl.BlockSpec((_BR, D), lambda k: (k, 0)),
    out_shape=jax.ShapeDtypeStruct((N, D), jnp.float32),
)


def kernel(feature, segment_ids):
    ids = segment_ids.astype(jnp.int32)
    ids_pad = jnp.concatenate(
        [ids, jnp.full((NPAD - N,), PAD_ID, jnp.int32)]
    )
    counts = _sc_counts(ids_pad)[:N]
    return _tc_scale(feature, counts[:, None])


# speedup vs baseline: 4.6450x; 1.0377x over previous
import functools

import jax
import jax.numpy as jnp
from jax import lax
from jax.experimental import pallas as pl
from jax.experimental.pallas import tpu as pltpu, tpu_sc as plsc

N = 50000
D = 256
NUM_GRAPHS = 512

NC = 2
NS = 16
NW = NC * NS
NPAD = 50176
CHUNK = NPAD // NW
CVECS = CHUNK // 16
HCHUNK = NPAD // NS
HVECS = HCHUNK // 16
BINS = 640
PAD_ID = NUM_GRAPHS

_sc_mesh = plsc.VectorSubcoreMesh(core_axis_name="c", subcore_axis_name="s")


@functools.partial(
    pl.kernel,
    out_type=jax.ShapeDtypeStruct((NPAD,), jnp.float32),
    mesh=_sc_mesh,
    scratch_types=[
        pltpu.VMEM((HCHUNK,), jnp.int32),
        pltpu.VMEM((CHUNK,), jnp.int32),
        pltpu.VMEM((BINS,), jnp.float32),
        pltpu.VMEM((BINS // 128, 128), jnp.int32),
        pltpu.VMEM((CHUNK,), jnp.float32),
        pltpu.VMEM_SHARED((BINS,), jnp.float32),
        pltpu.SemaphoreType.DMA,
    ],
    compiler_params=pltpu.CompilerParams(needs_layout_passes=False),
)
def _sc_counts(ids_hbm, out_hbm, idxh_v, idxo_v, hist_v, binidx_v, cnt_v,
               shared, sem):
    cid = lax.axis_index("c")
    sid = lax.axis_index("s")
    wid = sid * NC + cid
    pltpu.sync_copy(ids_hbm.at[pl.ds(sid * HCHUNK, HCHUNK)], idxh_v)
    pltpu.sync_copy(ids_hbm.at[pl.ds(wid * CHUNK, CHUNK)], idxo_v)

    zeros16 = jnp.zeros((16,), jnp.float32)
    ones16 = jnp.ones((16,), jnp.float32)
    iota16 = lax.iota(jnp.int32, 16)
    for j in range(BINS // 16):
        hist_v[pl.ds(j * 16, 16)] = zeros16
    for c in range(BINS // 128):
        for k in range(8):
            binidx_v[c, pl.ds(k * 16, 16)] = iota16 + (c * 128 + k * 16)

    @pl.when(sid == 0)
    def _():
        pltpu.sync_copy(hist_v, shared)

    for i in range(HVECS):
        v = idxh_v[pl.ds(i * 16, 16)]
        plsc.addupdate_scatter(hist_v, [v], ones16)

    plsc.subcore_barrier()
    for c in range(BINS // 128):
        pltpu.sync_copy(
            hist_v.at[pl.ds(c * 128, 128)],
            shared.at[binidx_v.at[c]],
            add=True,
        )
    plsc.subcore_barrier()
    pltpu.sync_copy(shared, hist_v)

    for i in range(CVECS):
        sl = pl.ds(i * 16, 16)
        cnt_v[sl] = plsc.load_gather(hist_v, [idxo_v[sl]])
    pltpu.sync_copy(cnt_v, out_hbm.at[pl.ds(wid * CHUNK, CHUNK)])


_BR = 5000


def _tc_scale_body(f_ref, c_ref, o_ref):
    o_ref[...] = f_ref[...] * lax.rsqrt(c_ref[...])


_tc_scale = pl.pallas_call(
    _tc_scale_body,
    grid=(N // _BR,),
    in_specs=[
        pl.BlockSpec((_BR, D), lambda k: (k, 0)),
        pl.BlockSpec((_BR, 1), lambda k: (k, 0)),
    ],
    out_specs=pl.BlockSpec((_BR, D), lambda k: (k, 0)),
    out_shape=jax.ShapeDtypeStruct((N, D), jnp.float32),
)


def kernel(feature, segment_ids):
    ids = segment_ids.astype(jnp.int32)
    ids_pad = jnp.concatenate(
        [ids, jnp.full((NPAD - N,), PAD_ID, jnp.int32)]
    )
    counts = _sc_counts(ids_pad)[:N]
    return _tc_scale(feature, counts[:, None])
